# baseline (speedup 1.0000x reference)
import functools

import jax
import jax.numpy as jnp
from jax.experimental import pallas as pl
from jax.experimental.pallas import tpu as pltpu

B = 1
N = 8192
FEAT_DIM = 6
NUM_CLASSES = 17
K = 32


def _swap_halves(x, d):
    n = x.shape[0]
    y = x.reshape((n // (2 * d), 2, d) + x.shape[1:])
    y = jnp.concatenate([y[:, 1:2], y[:, 0:1]], axis=1)
    return y.reshape(x.shape)


def _ce0(v, ix, pos, s, m):
    d = 1 << s
    pv = _swap_halves(v, d)
    pi = _swap_halves(ix, d)
    less = (v < pv) | ((v == pv) & (ix < pi))
    low = (pos & d) == 0
    keep = less ^ (~low)
    if m is not None:
        keep = keep ^ ((pos & (1 << m)) != 0)
    return jnp.where(keep, v, pv), jnp.where(keep, ix, pi)


def _iota0(shape):
    return jax.lax.broadcasted_iota(jnp.int32, shape, 0)


def _dist_topk_kernel(a_ref, at_ref, out_ref, v_ref, ix_ref, *, k):
    a = a_ref[...]
    at = at_ref[...]
    dot = jnp.dot(a, at, preferred_element_type=jnp.float32)
    n2r = jnp.sum(a * a, axis=1, keepdims=True)
    n2c = jnp.sum(at * at, axis=0, keepdims=True)
    v_ref[...] = jnp.maximum(n2r - 2.0 * dot + n2c, 0.0)
    ix_ref[...] = _iota0(v_ref.shape)

    W = v_ref.shape[0]
    lk = k.bit_length() - 1
    CH = 2048

    def stage(w, s, m):
        sh = v_ref.shape[1:]
        for base in range(0, w, min(CH, w)):
            ch = min(CH, w)
            v, ix = v_ref[base:base + ch], ix_ref[base:base + ch]
            pos = _iota0((ch,) + sh) + base
            v2, i2 = _ce0(v, ix, pos, s, m)
            v_ref[base:base + ch], ix_ref[base:base + ch] = v2, i2

    def reduce_pairs(w):
        for g in range(w // (2 * k)):
            rb, ob = g * 2 * k, g * k
            for c0 in range(0, k, CH):
                ch = min(CH, k)
                lo_v = v_ref[rb + c0:rb + c0 + ch]
                hi_v = v_ref[rb + k + c0:rb + k + c0 + ch]
                lo_i = ix_ref[rb + c0:rb + c0 + ch]
                hi_i = ix_ref[rb + k + c0:rb + k + c0 + ch]
                less = (lo_v < hi_v) | ((lo_v == hi_v) & (lo_i < hi_i))
                v_ref[ob + c0:ob + c0 + ch] = jnp.where(less, lo_v, hi_v)
                ix_ref[ob + c0:ob + c0 + ch] = jnp.where(less, lo_i, hi_i)

    for m in range(1, lk + 1):
        for s in range(m - 1, -1, -1):
            stage(W, s, m)
    w = W
    while w > k:
        reduce_pairs(w)
        w //= 2
        m = None if w == k else lk
        for s in range(lk - 1, -1, -1):
            stage(w, s, m)
    out_ref[...] = ix_ref[:k]


def _dist_topk(feat, k, lanes=128):
    n, c = feat.shape
    cpad = max(8, -(-c // 8) * 8)
    a = jnp.zeros((n, cpad), jnp.float32).at[:, :c].set(feat)
    at = a.T
    out = pl.pallas_call(
        functools.partial(_dist_topk_kernel, k=k),
        grid=(n // lanes,),
        in_specs=[
            pl.BlockSpec((n, cpad), lambda i: (0, 0)),
            pl.BlockSpec((cpad, lanes), lambda i: (0, i)),
        ],
        out_specs=pl.BlockSpec((k, lanes), lambda i: (0, i)),
        out_shape=jax.ShapeDtypeStruct((k, n), jnp.int32),
        scratch_shapes=[
            pltpu.VMEM((n, lanes), jnp.float32),
            pltpu.VMEM((n, lanes), jnp.int32),
        ],
        compiler_params=pltpu.CompilerParams(
            dimension_semantics=("parallel",),
        ),
    )(a, at)
    return out


def _gather(x, idx):
    return jax.vmap(lambda xb, ib: xb[ib])(x, idx)


def _edge_conv(x, idx, w1, b1, w2, b2):
    nbr = _gather(x, idx)
    ctr = jnp.broadcast_to(x[:, :, None, :], nbr.shape)
    e = jnp.concatenate([nbr - ctr, ctr], axis=-1)
    h = jax.nn.relu(e @ w1 + b1)
    h = jax.nn.relu(h @ w2 + b2)
    return jnp.max(h, axis=2)


def _stn(x, p):
    h = jax.nn.relu(x @ p["stn_c1_w"] + p["stn_c1_b"])
    h = jax.nn.relu(h @ p["stn_c2_w"] + p["stn_c2_b"])
    h = jax.nn.relu(h @ p["stn_c3_w"] + p["stn_c3_b"])
    g = jnp.max(h, axis=1)
    g = jax.nn.relu(g @ p["stn_f1_w"] + p["stn_f1_b"])
    g = jax.nn.relu(g @ p["stn_f2_w"] + p["stn_f2_b"])
    t = g @ p["stn_f3_w"] + p["stn_f3_b"]
    k = x.shape[-1]
    t = t.reshape(-1, k, k) + jnp.eye(k, dtype=x.dtype)
    return jnp.einsum("bnc,bcd->bnd", x, t)


def _tail_kernel(x_ref, gh_w, gh_b, fi_w, fi_b,
                 r1m1_w, r1m1_b, r1m2_w, r1m2_b, r1sc_w, r1sc_b,
                 r2m1_w, r2m1_b, r2m2_w, r2m2_b, r2sc_w, r2sc_b,
                 ow, ob, out_ref):
    x = x_ref[...]
    x = jax.nn.relu(x @ gh_w[...] + gh_b[...])
    x = x * jax.nn.sigmoid(x @ fi_w[...] + fi_b[...])
    h = jax.nn.relu(x @ r1m1_w[...] + r1m1_b[...])
    h = jax.nn.relu(h @ r1m2_w[...] + r1m2_b[...])
    x = h + (x @ r1sc_w[...] + r1sc_b[...])
    h = jax.nn.relu(x @ r2m1_w[...] + r2m1_b[...])
    h = jax.nn.relu(h @ r2m2_w[...] + r2m2_b[...])
    x = h + (x @ r2sc_w[...] + r2sc_b[...])
    out_ref[...] = x @ ow[...] + ob[...]


def _dense_tail(x, p):
    ow = jnp.concatenate([p["out_w"], p["edge_w"]], axis=1)
    ob = jnp.concatenate([p["out_b"], p["edge_b"]], axis=0)
    blk = 1024
    full = lambda shape: pl.BlockSpec(shape, lambda i: (0,) * len(shape))
    args = [
        p["gh_w"], p["gh_b"], p["fi_w"], p["fi_b"],
        p["rb1_m1_w"], p["rb1_m1_b"], p["rb1_m2_w"], p["rb1_m2_b"],
        p["rb1_sc_w"], p["rb1_sc_b"],
        p["rb2_m1_w"], p["rb2_m1_b"], p["rb2_m2_w"], p["rb2_m2_b"],
        p["rb2_sc_w"], p["rb2_sc_b"], ow, ob,
    ]
    in_specs = [pl.BlockSpec((blk, 240), lambda i: (i, 0))]
    in_specs += [full(a.shape) for a in args]
    out = pl.pallas_call(
        _tail_kernel,
        grid=(N // blk,),
        in_specs=in_specs,
        out_specs=pl.BlockSpec((blk, NUM_CLASSES + 2), lambda i: (i, 0)),
        out_shape=jax.ShapeDtypeStruct((N, NUM_CLASSES + 2), jnp.float32),
        compiler_params=pltpu.CompilerParams(
            dimension_semantics=("parallel",),
        ),
    )(x, *args)
    return out[:, :NUM_CLASSES], out[:, NUM_CLASSES:]


def kernel(data, params):
    p = params
    x = jnp.transpose(data, (0, 2, 1))
    pos = x[0, :, :3]
    x = _stn(x, p)

    order = _dist_topk(pos, 2048)
    idx_knn1 = order[:K].T[None]
    idx_d1 = order[: 200 : 200 // K][:K].T[None]
    idx_d2 = order[: 900 : 900 // K][:K].T[None]
    idx_d3 = order[: 1800 : 1800 // K][:K].T[None]

    x1 = _edge_conv(x, idx_knn1, p["eg1_w1"], p["eg1_b1"], p["eg1_w2"], p["eg1_b2"])
    idx_knn2 = _dist_topk(x1[0], K).T[None]
    x2 = _edge_conv(x1, idx_knn2, p["eg2_w1"], p["eg2_b1"], p["eg2_w2"], p["eg2_b2"])
    idx_knn3 = _dist_topk(x2[0], K).T[None]
    x3 = _edge_conv(x2, idx_knn3, p["eg3_w1"], p["eg3_b1"], p["eg3_w2"], p["eg3_b2"])
    x = jnp.concatenate([x1, x2, x3], axis=2)
    x = jax.nn.relu(x @ p["lh_w"] + p["lh_b"])
    d1 = _edge_conv(x, idx_d1, p["dg1_w1"], p["dg1_b1"], p["dg1_w2"], p["dg1_b2"])
    d2 = _edge_conv(d1, idx_d2, p["dg2_w1"], p["dg2_b1"], p["dg2_w2"], p["dg2_b2"])
    d3 = _edge_conv(d2, idx_d3, p["dg3_w1"], p["dg3_b1"], p["dg3_w2"], p["dg3_b2"])
    x = jnp.concatenate([x, d1, d2, d3], axis=2)

    seg, edge = _dense_tail(x[0], p)
    return (seg.T[None], edge.T[None])


# speedup vs baseline: 1.0507x; 1.0507x over previous
import functools

import jax
import jax.numpy as jnp
from jax.experimental import pallas as pl
from jax.experimental.pallas import tpu as pltpu

B = 1
N = 8192
FEAT_DIM = 6
NUM_CLASSES = 17
K = 32


def _ce0(v, ix, pos, s, m):
    d = 1 << s
    n = v.shape[0]
    low = (pos & d) == 0
    pv = jnp.where(low, pltpu.roll(v, n - d, 0), pltpu.roll(v, d, 0))
    pi = jnp.where(low, pltpu.roll(ix, n - d, 0), pltpu.roll(ix, d, 0))
    less = (v < pv) | ((v == pv) & (ix < pi))
    keep = less ^ (~low)
    if m is not None:
        keep = keep ^ ((pos & (1 << m)) != 0)
    return jnp.where(keep, v, pv), jnp.where(keep, ix, pi)


def _iota0(shape):
    return jax.lax.broadcasted_iota(jnp.int32, shape, 0)


def _dist_topk_kernel(a_ref, at_ref, out_ref, v_ref, ix_ref, *, k):
    a = a_ref[...]
    at = at_ref[...]
    dot = jnp.dot(a, at, preferred_element_type=jnp.float32)
    n2r = jnp.sum(a * a, axis=1, keepdims=True)
    n2c = jnp.sum(at * at, axis=0, keepdims=True)
    v_ref[...] = jnp.maximum(n2r - 2.0 * dot + n2c, 0.0)
    ix_ref[...] = _iota0(v_ref.shape)

    W = v_ref.shape[0]
    lk = k.bit_length() - 1
    CH = 2048

    def stage(w, s, m):
        sh = v_ref.shape[1:]
        for base in range(0, w, min(CH, w)):
            ch = min(CH, w)
            v, ix = v_ref[base:base + ch], ix_ref[base:base + ch]
            pos = _iota0((ch,) + sh) + base
            v2, i2 = _ce0(v, ix, pos, s, m)
            v_ref[base:base + ch], ix_ref[base:base + ch] = v2, i2

    def reduce_pairs(w):
        for g in range(w // (2 * k)):
            rb, ob = g * 2 * k, g * k
            for c0 in range(0, k, CH):
                ch = min(CH, k)
                lo_v = v_ref[rb + c0:rb + c0 + ch]
                hi_v = v_ref[rb + k + c0:rb + k + c0 + ch]
                lo_i = ix_ref[rb + c0:rb + c0 + ch]
                hi_i = ix_ref[rb + k + c0:rb + k + c0 + ch]
                less = (lo_v < hi_v) | ((lo_v == hi_v) & (lo_i < hi_i))
                v_ref[ob + c0:ob + c0 + ch] = jnp.where(less, lo_v, hi_v)
                ix_ref[ob + c0:ob + c0 + ch] = jnp.where(less, lo_i, hi_i)

    for m in range(1, lk + 1):
        for s in range(m - 1, -1, -1):
            stage(W, s, m)
    w = W
    while w > k:
        reduce_pairs(w)
        w //= 2
        m = None if w == k else lk
        for s in range(lk - 1, -1, -1):
            stage(w, s, m)
    out_ref[...] = ix_ref[:k]


def _dist_topk(feat, k, lanes=128):
    n, c = feat.shape
    cpad = max(8, -(-c // 8) * 8)
    a = jnp.zeros((n, cpad), jnp.float32).at[:, :c].set(feat)
    at = a.T
    out = pl.pallas_call(
        functools.partial(_dist_topk_kernel, k=k),
        grid=(n // lanes,),
        in_specs=[
            pl.BlockSpec((n, cpad), lambda i: (0, 0)),
            pl.BlockSpec((cpad, lanes), lambda i: (0, i)),
        ],
        out_specs=pl.BlockSpec((k, lanes), lambda i: (0, i)),
        out_shape=jax.ShapeDtypeStruct((k, n), jnp.int32),
        scratch_shapes=[
            pltpu.VMEM((n, lanes), jnp.float32),
            pltpu.VMEM((n, lanes), jnp.int32),
        ],
        compiler_params=pltpu.CompilerParams(
            dimension_semantics=("parallel",),
        ),
    )(a, at)
    return out


def _gather(x, idx):
    return jax.vmap(lambda xb, ib: xb[ib])(x, idx)


def _edge_conv(x, idx, w1, b1, w2, b2):
    nbr = _gather(x, idx)
    ctr = jnp.broadcast_to(x[:, :, None, :], nbr.shape)
    e = jnp.concatenate([nbr - ctr, ctr], axis=-1)
    h = jax.nn.relu(e @ w1 + b1)
    h = jax.nn.relu(h @ w2 + b2)
    return jnp.max(h, axis=2)


def _stn(x, p):
    h = jax.nn.relu(x @ p["stn_c1_w"] + p["stn_c1_b"])
    h = jax.nn.relu(h @ p["stn_c2_w"] + p["stn_c2_b"])
    h = jax.nn.relu(h @ p["stn_c3_w"] + p["stn_c3_b"])
    g = jnp.max(h, axis=1)
    g = jax.nn.relu(g @ p["stn_f1_w"] + p["stn_f1_b"])
    g = jax.nn.relu(g @ p["stn_f2_w"] + p["stn_f2_b"])
    t = g @ p["stn_f3_w"] + p["stn_f3_b"]
    k = x.shape[-1]
    t = t.reshape(-1, k, k) + jnp.eye(k, dtype=x.dtype)
    return jnp.einsum("bnc,bcd->bnd", x, t)


def _tail_kernel(x_ref, gh_w, gh_b, fi_w, fi_b,
                 r1m1_w, r1m1_b, r1m2_w, r1m2_b, r1sc_w, r1sc_b,
                 r2m1_w, r2m1_b, r2m2_w, r2m2_b, r2sc_w, r2sc_b,
                 ow, ob, out_ref):
    x = x_ref[...]
    x = jax.nn.relu(x @ gh_w[...] + gh_b[...])
    x = x * jax.nn.sigmoid(x @ fi_w[...] + fi_b[...])
    h = jax.nn.relu(x @ r1m1_w[...] + r1m1_b[...])
    h = jax.nn.relu(h @ r1m2_w[...] + r1m2_b[...])
    x = h + (x @ r1sc_w[...] + r1sc_b[...])
    h = jax.nn.relu(x @ r2m1_w[...] + r2m1_b[...])
    h = jax.nn.relu(h @ r2m2_w[...] + r2m2_b[...])
    x = h + (x @ r2sc_w[...] + r2sc_b[...])
    out_ref[...] = x @ ow[...] + ob[...]


def _dense_tail(x, p):
    ow = jnp.concatenate([p["out_w"], p["edge_w"]], axis=1)
    ob = jnp.concatenate([p["out_b"], p["edge_b"]], axis=0)
    blk = 1024
    full = lambda shape: pl.BlockSpec(shape, lambda i: (0,) * len(shape))
    args = [
        p["gh_w"], p["gh_b"], p["fi_w"], p["fi_b"],
        p["rb1_m1_w"], p["rb1_m1_b"], p["rb1_m2_w"], p["rb1_m2_b"],
        p["rb1_sc_w"], p["rb1_sc_b"],
        p["rb2_m1_w"], p["rb2_m1_b"], p["rb2_m2_w"], p["rb2_m2_b"],
        p["rb2_sc_w"], p["rb2_sc_b"], ow, ob,
    ]
    in_specs = [pl.BlockSpec((blk, 240), lambda i: (i, 0))]
    in_specs += [full(a.shape) for a in args]
    out = pl.pallas_call(
        _tail_kernel,
        grid=(N // blk,),
        in_specs=in_specs,
        out_specs=pl.BlockSpec((blk, NUM_CLASSES + 2), lambda i: (i, 0)),
        out_shape=jax.ShapeDtypeStruct((N, NUM_CLASSES + 2), jnp.float32),
        compiler_params=pltpu.CompilerParams(
            dimension_semantics=("parallel",),
        ),
    )(x, *args)
    return out[:, :NUM_CLASSES], out[:, NUM_CLASSES:]


def kernel(data, params):
    p = params
    x = jnp.transpose(data, (0, 2, 1))
    pos = x[0, :, :3]
    x = _stn(x, p)

    order = _dist_topk(pos, 2048)
    idx_knn1 = order[:K].T[None]
    idx_d1 = order[: 200 : 200 // K][:K].T[None]
    idx_d2 = order[: 900 : 900 // K][:K].T[None]
    idx_d3 = order[: 1800 : 1800 // K][:K].T[None]

    x1 = _edge_conv(x, idx_knn1, p["eg1_w1"], p["eg1_b1"], p["eg1_w2"], p["eg1_b2"])
    idx_knn2 = _dist_topk(x1[0], K).T[None]
    x2 = _edge_conv(x1, idx_knn2, p["eg2_w1"], p["eg2_b1"], p["eg2_w2"], p["eg2_b2"])
    idx_knn3 = _dist_topk(x2[0], K).T[None]
    x3 = _edge_conv(x2, idx_knn3, p["eg3_w1"], p["eg3_b1"], p["eg3_w2"], p["eg3_b2"])
    x = jnp.concatenate([x1, x2, x3], axis=2)
    x = jax.nn.relu(x @ p["lh_w"] + p["lh_b"])
    d1 = _edge_conv(x, idx_d1, p["dg1_w1"], p["dg1_b1"], p["dg1_w2"], p["dg1_b2"])
    d2 = _edge_conv(d1, idx_d2, p["dg2_w1"], p["dg2_b1"], p["dg2_w2"], p["dg2_b2"])
    d3 = _edge_conv(d2, idx_d3, p["dg3_w1"], p["dg3_b1"], p["dg3_w2"], p["dg3_b2"])
    x = jnp.concatenate([x, d1, d2, d3], axis=2)

    seg, edge = _dense_tail(x[0], p)
    return (seg.T[None], edge.T[None])


# speedup vs baseline: 2.2075x; 2.1009x over previous
import functools

import jax
import jax.numpy as jnp
from jax.experimental import pallas as pl
from jax.experimental.pallas import tpu as pltpu
from jax.experimental.pallas import tpu_sc as plsc

B = 1
N = 8192
FEAT_DIM = 6
NUM_CLASSES = 17
K = 32


def _ce0(v, ix, pos, s, m):
    d = 1 << s
    n = v.shape[0]
    low = (pos & d) == 0
    pv = jnp.where(low, pltpu.roll(v, n - d, 0), pltpu.roll(v, d, 0))
    pi = jnp.where(low, pltpu.roll(ix, n - d, 0), pltpu.roll(ix, d, 0))
    less = (v < pv) | ((v == pv) & (ix < pi))
    keep = less ^ (~low)
    if m is not None:
        keep = keep ^ ((pos & (1 << m)) != 0)
    return jnp.where(keep, v, pv), jnp.where(keep, ix, pi)


def _iota0(shape):
    return jax.lax.broadcasted_iota(jnp.int32, shape, 0)


def _dist_topk_kernel(a_ref, at_ref, out_ref, v_ref, ix_ref, *, k):
    a = a_ref[...]
    at = at_ref[...]
    dot = jnp.dot(a, at, preferred_element_type=jnp.float32)
    n2r = jnp.sum(a * a, axis=1, keepdims=True)
    n2c = jnp.sum(at * at, axis=0, keepdims=True)
    v_ref[...] = jnp.maximum(n2r - 2.0 * dot + n2c, 0.0)
    ix_ref[...] = _iota0(v_ref.shape)

    W = v_ref.shape[0]
    lk = k.bit_length() - 1
    CH = 2048

    def stage(w, s, m):
        sh = v_ref.shape[1:]
        for base in range(0, w, min(CH, w)):
            ch = min(CH, w)
            v, ix = v_ref[base:base + ch], ix_ref[base:base + ch]
            pos = _iota0((ch,) + sh) + base
            v2, i2 = _ce0(v, ix, pos, s, m)
            v_ref[base:base + ch], ix_ref[base:base + ch] = v2, i2

    def reduce_pairs(w):
        for g in range(w // (2 * k)):
            rb, ob = g * 2 * k, g * k
            for c0 in range(0, k, CH):
                ch = min(CH, k)
                lo_v = v_ref[rb + c0:rb + c0 + ch]
                hi_v = v_ref[rb + k + c0:rb + k + c0 + ch]
                lo_i = ix_ref[rb + c0:rb + c0 + ch]
                hi_i = ix_ref[rb + k + c0:rb + k + c0 + ch]
                less = (lo_v < hi_v) | ((lo_v == hi_v) & (lo_i < hi_i))
                v_ref[ob + c0:ob + c0 + ch] = jnp.where(less, lo_v, hi_v)
                ix_ref[ob + c0:ob + c0 + ch] = jnp.where(less, lo_i, hi_i)

    for m in range(1, lk + 1):
        for s in range(m - 1, -1, -1):
            stage(W, s, m)
    w = W
    while w > k:
        reduce_pairs(w)
        w //= 2
        m = None if w == k else lk
        for s in range(lk - 1, -1, -1):
            stage(w, s, m)
    out_ref[...] = ix_ref[:k]


def _dist_topk(feat, k, lanes=128):
    n, c = feat.shape
    cpad = max(8, -(-c // 8) * 8)
    a = jnp.zeros((n, cpad), jnp.float32).at[:, :c].set(feat)
    at = a.T
    out = pl.pallas_call(
        functools.partial(_dist_topk_kernel, k=k),
        grid=(n // lanes,),
        in_specs=[
            pl.BlockSpec((n, cpad), lambda i: (0, 0)),
            pl.BlockSpec((cpad, lanes), lambda i: (0, i)),
        ],
        out_specs=pl.BlockSpec((k, lanes), lambda i: (0, i)),
        out_shape=jax.ShapeDtypeStruct((k, n), jnp.int32),
        scratch_shapes=[
            pltpu.VMEM((n, lanes), jnp.float32),
            pltpu.VMEM((n, lanes), jnp.int32),
        ],
        compiler_params=pltpu.CompilerParams(
            dimension_semantics=("parallel",),
        ),
    )(a, at)
    return out


_SC_MESH = plsc.VectorSubcoreMesh(core_axis_name="core", subcore_axis_name="subcore")
_GW = 128


def _sc_gather(x, idx_flat):
    m = idx_flat.shape[0]
    c = x.shape[1]
    idx2 = idx_flat.reshape(1, m)

    @functools.partial(pl.kernel,
                       out_type=jax.ShapeDtypeStruct((m, c), x.dtype),
                       mesh=_SC_MESH)
    def _kern(x_hbm, i_hbm, o_hbm):
        def body(i_vmem, o_vmem):
            pltpu.sync_copy(x_hbm.at[i_vmem.at[0]], o_vmem)

        pltpu.emit_pipeline(
            body,
            grid=(m // _GW,),
            in_specs=[pl.BlockSpec((1, _GW), index_map=lambda i: (0, i))],
            out_specs=[pl.BlockSpec((_GW, c), index_map=lambda i: (i, 0))],
            core_axis_name="subcore",
            dimension_semantics=(pltpu.PARALLEL,),
        )(i_hbm, o_hbm)

    return _kern(x, idx2)


def _gather(x, idx):
    n, k = idx.shape[1], idx.shape[2]
    c = x.shape[2]
    xp = jnp.pad(x[0], ((0, 0), (0, 128 - c)))
    flat = idx[0].reshape(n * k)
    return _sc_gather(xp, flat)[:, :c].reshape(1, n, k, c)


def _edge_conv(x, idx, w1, b1, w2, b2):
    nbr = _gather(x, idx)
    ctr = jnp.broadcast_to(x[:, :, None, :], nbr.shape)
    e = jnp.concatenate([nbr - ctr, ctr], axis=-1)
    h = jax.nn.relu(e @ w1 + b1)
    h = jax.nn.relu(h @ w2 + b2)
    return jnp.max(h, axis=2)


def _stn(x, p):
    h = jax.nn.relu(x @ p["stn_c1_w"] + p["stn_c1_b"])
    h = jax.nn.relu(h @ p["stn_c2_w"] + p["stn_c2_b"])
    h = jax.nn.relu(h @ p["stn_c3_w"] + p["stn_c3_b"])
    g = jnp.max(h, axis=1)
    g = jax.nn.relu(g @ p["stn_f1_w"] + p["stn_f1_b"])
    g = jax.nn.relu(g @ p["stn_f2_w"] + p["stn_f2_b"])
    t = g @ p["stn_f3_w"] + p["stn_f3_b"]
    k = x.shape[-1]
    t = t.reshape(-1, k, k) + jnp.eye(k, dtype=x.dtype)
    return jnp.einsum("bnc,bcd->bnd", x, t)


def _tail_kernel(x_ref, gh_w, gh_b, fi_w, fi_b,
                 r1m1_w, r1m1_b, r1m2_w, r1m2_b, r1sc_w, r1sc_b,
                 r2m1_w, r2m1_b, r2m2_w, r2m2_b, r2sc_w, r2sc_b,
                 ow, ob, out_ref):
    x = x_ref[...]
    x = jax.nn.relu(x @ gh_w[...] + gh_b[...])
    x = x * jax.nn.sigmoid(x @ fi_w[...] + fi_b[...])
    h = jax.nn.relu(x @ r1m1_w[...] + r1m1_b[...])
    h = jax.nn.relu(h @ r1m2_w[...] + r1m2_b[...])
    x = h + (x @ r1sc_w[...] + r1sc_b[...])
    h = jax.nn.relu(x @ r2m1_w[...] + r2m1_b[...])
    h = jax.nn.relu(h @ r2m2_w[...] + r2m2_b[...])
    x = h + (x @ r2sc_w[...] + r2sc_b[...])
    out_ref[...] = x @ ow[...] + ob[...]


def _dense_tail(x, p):
    ow = jnp.concatenate([p["out_w"], p["edge_w"]], axis=1)
    ob = jnp.concatenate([p["out_b"], p["edge_b"]], axis=0)
    blk = 1024
    full = lambda shape: pl.BlockSpec(shape, lambda i: (0,) * len(shape))
    args = [
        p["gh_w"], p["gh_b"], p["fi_w"], p["fi_b"],
        p["rb1_m1_w"], p["rb1_m1_b"], p["rb1_m2_w"], p["rb1_m2_b"],
        p["rb1_sc_w"], p["rb1_sc_b"],
        p["rb2_m1_w"], p["rb2_m1_b"], p["rb2_m2_w"], p["rb2_m2_b"],
        p["rb2_sc_w"], p["rb2_sc_b"], ow, ob,
    ]
    in_specs = [pl.BlockSpec((blk, 240), lambda i: (i, 0))]
    in_specs += [full(a.shape) for a in args]
    out = pl.pallas_call(
        _tail_kernel,
        grid=(N // blk,),
        in_specs=in_specs,
        out_specs=pl.BlockSpec((blk, NUM_CLASSES + 2), lambda i: (i, 0)),
        out_shape=jax.ShapeDtypeStruct((N, NUM_CLASSES + 2), jnp.float32),
        compiler_params=pltpu.CompilerParams(
            dimension_semantics=("parallel",),
        ),
    )(x, *args)
    return out[:, :NUM_CLASSES], out[:, NUM_CLASSES:]


def kernel(data, params):
    p = params
    x = jnp.transpose(data, (0, 2, 1))
    pos = x[0, :, :3]
    x = _stn(x, p)

    order = _dist_topk(pos, 2048)
    idx_knn1 = order[:K].T[None]
    idx_d1 = order[: 200 : 200 // K][:K].T[None]
    idx_d2 = order[: 900 : 900 // K][:K].T[None]
    idx_d3 = order[: 1800 : 1800 // K][:K].T[None]

    x1 = _edge_conv(x, idx_knn1, p["eg1_w1"], p["eg1_b1"], p["eg1_w2"], p["eg1_b2"])
    idx_knn2 = _dist_topk(x1[0], K).T[None]
    x2 = _edge_conv(x1, idx_knn2, p["eg2_w1"], p["eg2_b1"], p["eg2_w2"], p["eg2_b2"])
    idx_knn3 = _dist_topk(x2[0], K).T[None]
    x3 = _edge_conv(x2, idx_knn3, p["eg3_w1"], p["eg3_b1"], p["eg3_w2"], p["eg3_b2"])
    x = jnp.concatenate([x1, x2, x3], axis=2)
    x = jax.nn.relu(x @ p["lh_w"] + p["lh_b"])
    d1 = _edge_conv(x, idx_d1, p["dg1_w1"], p["dg1_b1"], p["dg1_w2"], p["dg1_b2"])
    d2 = _edge_conv(d1, idx_d2, p["dg2_w1"], p["dg2_b1"], p["dg2_w2"], p["dg2_b2"])
    d3 = _edge_conv(d2, idx_d3, p["dg3_w1"], p["dg3_b1"], p["dg3_w2"], p["dg3_b2"])
    x = jnp.concatenate([x, d1, d2, d3], axis=2)

    seg, edge = _dense_tail(x[0], p)
    return (seg.T[None], edge.T[None])
